# baseline (speedup 1.0000x reference)
import functools

import jax
import jax.numpy as jnp
from jax import lax
from jax.experimental import pallas as pl
from jax.experimental.pallas import tpu as pltpu
from jax.experimental.pallas import tpu_sc as plsc

OUT = 7
SR = 2
SCALE = 0.25
S2 = OUT * OUT * SR * SR * 4
NBIN = OUT * OUT
KPB = SR * SR * 4
BIN0 = 25
ROWS0 = BIN0 * KPB
ROWS1 = S2 - ROWS0


def _index_weight_body(rois_ref, idx_ref, w_ref, *, H, W, C):
    bm = rois_ref.shape[0]
    shape = (bm, S2)
    l = lax.broadcasted_iota(jnp.int32, shape, 1)
    bin_ = l // KPB
    k = l % KPB
    sub = k // 4
    corner = k % 4
    by = bin_ // OUT
    bx = bin_ % OUT
    sy = sub // SR
    sx = sub % SR
    cy = corner // 2
    cx = corner % 2
    iy = (by * SR + sy).astype(jnp.float32)
    ix = (bx * SR + sx).astype(jnp.float32)

    b = rois_ref[:, 0:1].astype(jnp.int32)
    rx0 = rois_ref[:, 1:2] * SCALE
    ry0 = rois_ref[:, 2:3] * SCALE
    rx1 = rois_ref[:, 3:4] * SCALE
    ry1 = rois_ref[:, 4:5] * SCALE
    roi_w = jnp.maximum(rx1 - rx0, 1.0)
    roi_h = jnp.maximum(ry1 - ry0, 1.0)
    bin_w = roi_w * (1.0 / OUT)
    bin_h = roi_h * (1.0 / OUT)

    ys = ry0 + (iy + 0.5) * (bin_h * (1.0 / SR))
    xs = rx0 + (ix + 0.5) * (bin_w * (1.0 / SR))
    ys = jnp.clip(ys, 0.0, H - 1.0)
    xs = jnp.clip(xs, 0.0, W - 1.0)
    y0f = jnp.floor(ys)
    x0f = jnp.floor(xs)
    y0 = y0f.astype(jnp.int32)
    x0 = x0f.astype(jnp.int32)
    ly = ys - y0f
    lx = xs - x0f
    y1 = jnp.minimum(y0 + 1, H - 1)
    x1 = jnp.minimum(x0 + 1, W - 1)
    ysel = jnp.where(cy == 1, y1, y0)
    xsel = jnp.where(cx == 1, x1, x0)
    wy = jnp.where(cy == 1, ly, 1.0 - ly)
    wx = jnp.where(cx == 1, lx, 1.0 - lx)
    idx_ref[...] = b * (H * W) + ysel * W + xsel
    w_ref[...] = wy * wx * (1.0 / (SR * SR))


def _make_index_kernel(M, H, W, C):
    bm = 200 if M % 200 == 0 else M
    grid = M // bm
    return pl.pallas_call(
        functools.partial(_index_weight_body, H=H, W=W, C=C),
        grid=(grid,),
        in_specs=[pl.BlockSpec((bm, 5), lambda i: (i, 0))],
        out_specs=[
            pl.BlockSpec((bm, S2), lambda i: (i, 0)),
            pl.BlockSpec((bm, S2), lambda i: (i, 0)),
        ],
        out_shape=[
            jax.ShapeDtypeStruct((M, S2), jnp.int32),
            jax.ShapeDtypeStruct((M, S2), jnp.float32),
        ],
    )


def _make_sc_gather(M, V, C):
    info = plsc.get_sparse_core_info()
    NC, NS = info.num_cores, info.num_subcores
    NW = NC * NS
    rois_per_w = (M + NW - 1) // NW
    R8 = C // 16
    mesh = plsc.VectorSubcoreMesh(core_axis_name="c", subcore_axis_name="s")

    @functools.partial(
        pl.kernel,
        mesh=mesh,
        out_type=jax.ShapeDtypeStruct((M * C * NBIN,), jnp.float32),
        scratch_types=[
            pltpu.VMEM((ROWS0,), jnp.int32),
            pltpu.VMEM((ROWS1,), jnp.int32),
            pltpu.VMEM((ROWS0,), jnp.int32),
            pltpu.VMEM((ROWS1,), jnp.int32),
            pltpu.VMEM((S2,), jnp.float32),
            pltpu.VMEM((S2,), jnp.float32),
            pltpu.VMEM((ROWS0, C), jnp.float32),
            pltpu.VMEM((ROWS1, C), jnp.float32),
            pltpu.VMEM((2 * NBIN * C,), jnp.float32),
            pltpu.SemaphoreType.DMA,
            pltpu.SemaphoreType.DMA,
            pltpu.SemaphoreType.DMA,
        ],
        compiler_params=pltpu.CompilerParams(needs_layout_passes=False),
    )
    def sc_kernel(xt_hbm, idx_hbm, w_hbm, out_hbm, i00, i01, i10, i11, w_v0,
                  w_v1, rows0, rows1, out_v, isem, osem, rsem):
        wid = lax.axis_index("s") * NC + lax.axis_index("c")
        idx_refs = ((i00, i01), (i10, i11))
        w_refs = (w_v0, w_v1)
        row_bufs = (rows0, rows1)
        starts = (0, ROWS0)
        sizes = (ROWS0, ROWS1)

        def idx_copies(m, p):
            base = pl.multiple_of(m * S2, 8)
            base1 = pl.multiple_of(m * S2 + ROWS0, 8)
            return (
                pltpu.make_async_copy(
                    idx_hbm.at[pl.ds(base, ROWS0)], idx_refs[p][0], isem
                ),
                pltpu.make_async_copy(
                    idx_hbm.at[pl.ds(base1, ROWS1)], idx_refs[p][1], isem
                ),
                pltpu.make_async_copy(
                    w_hbm.at[pl.ds(base, S2)], w_refs[p], isem
                ),
            )

        def row_stream(p, h):
            return pltpu.make_async_copy(
                xt_hbm.at[idx_refs[p][h]], row_bufs[h], rsem
            )

        def out_copy(p, m):
            obase = pl.multiple_of(m * (C * NBIN), 8)
            return pltpu.make_async_copy(
                out_v.at[pl.ds(p * (C * NBIN), C * NBIN)],
                out_hbm.at[pl.ds(obase, C * NBIN)],
                osem,
            )

        lane_off = lax.iota(jnp.int32, 16) * NBIN

        def compute_half(p, h):
            nbins = sizes[h] // KPB
            bin0 = starts[h] // KPB
            buf = row_bufs[h]

            def per_bin(b_, carry2):
                gbin = bin0 + b_
                wvec = w_refs[p][pl.ds(gbin * KPB, KPB)]
                base = b_ * KPB
                accs = [jnp.zeros((16,), jnp.float32) for _ in range(R8)]
                for k in range(KPB):
                    ws = wvec[k]
                    for r in range(R8):
                        accs[r] = accs[r] + ws * buf[base + k, pl.ds(r * 16, 16)]
                pbase = p * (C * NBIN) + gbin
                for r in range(R8):
                    plsc.store_scatter(
                        out_v, [lane_off + (pbase + 16 * r * NBIN)], accs[r]
                    )
                return carry2

            lax.fori_loop(0, nbins, per_bin, 0)

        def per_roi(j, p):
            m = j * NW + wid

            @pl.when(m < M)
            def _():
                row_stream(p, 0).wait()
                row_stream(p, 1).start()

                @pl.when(m + NW < M)
                def _():
                    for cp in idx_copies(m + NW, 1 - p):
                        cp.wait()

                @pl.when(j >= 2)
                def _():
                    out_copy(p, m).wait()

                compute_half(p, 0)
                row_stream(p, 1).wait()

                @pl.when(m + NW < M)
                def _():
                    row_stream(1 - p, 0).start()

                compute_half(p, 1)

                @pl.when(m + 2 * NW < M)
                def _():
                    for cp in idx_copies(m + 2 * NW, p):
                        cp.start()

                out_copy(p, m).start()

        def pair_body(jj, carry):
            per_roi(jj * 2, 0)
            per_roi(jj * 2 + 1, 1)
            return carry

        base0 = pl.multiple_of(wid * S2, 8)
        base0b = pl.multiple_of(wid * S2 + ROWS0, 8)
        pltpu.sync_copy(idx_hbm.at[pl.ds(base0, ROWS0)], idx_refs[0][0])
        pltpu.sync_copy(idx_hbm.at[pl.ds(base0b, ROWS1)], idx_refs[0][1])
        pltpu.sync_copy(w_hbm.at[pl.ds(base0, S2)], w_refs[0])
        row_stream(0, 0).start()

        @pl.when(wid + NW < M)
        def _():
            for cp in idx_copies(wid + NW, 1):
                cp.start()

        lax.fori_loop(0, (rois_per_w + 1) // 2, pair_body, 0)

        nj = (M - 1 - wid) // NW + 1
        for t in range(2):
            @pl.when(nj > t)
            def _():
                pltpu.make_async_copy(
                    out_v.at[pl.ds(0, C * NBIN)],
                    out_hbm.at[pl.ds(0, C * NBIN)],
                    osem,
                ).wait()

    return sc_kernel


def kernel(x, rois):
    B, C, H, W = x.shape
    M = rois.shape[0]
    xt = jnp.transpose(x, (0, 2, 3, 1)).reshape(B * H * W, C)
    idx, w = _make_index_kernel(M, H, W, C)(rois)
    out = _make_sc_gather(M, B * H * W, C)(
        xt, idx.reshape(M * S2), w.reshape(M * S2)
    )
    return out.reshape(M, C, OUT, OUT)


# speedup vs baseline: 2.0348x; 2.0348x over previous
import functools

import jax
import jax.numpy as jnp
from jax import lax
from jax.experimental import pallas as pl
from jax.experimental.pallas import tpu as pltpu
from jax.experimental.pallas import tpu_sc as plsc

OUT = 7
SR = 2
SCALE = 0.25
NBIN = OUT * OUT
RPB = SR * SR * 2
NROW = NBIN * RPB
KPB = RPB * 2
NWT = NBIN * KPB
BIN0 = 25
ROWS0 = BIN0 * RPB
ROWS1 = NROW - ROWS0


def _index_weight_body(rois_ref, idx_ref, w_ref, *, H, W, C):
    bm = rois_ref.shape[0]

    b = rois_ref[:, 0:1].astype(jnp.int32)
    rx0 = rois_ref[:, 1:2] * SCALE
    ry0 = rois_ref[:, 2:3] * SCALE
    rx1 = rois_ref[:, 3:4] * SCALE
    ry1 = rois_ref[:, 4:5] * SCALE
    roi_w = jnp.maximum(rx1 - rx0, 1.0)
    roi_h = jnp.maximum(ry1 - ry0, 1.0)
    bin_w = roi_w * (1.0 / OUT)
    bin_h = roi_h * (1.0 / OUT)

    def sample(iy, ix):
        ys = ry0 + (iy + 0.5) * (bin_h * (1.0 / SR))
        xs = rx0 + (ix + 0.5) * (bin_w * (1.0 / SR))
        ys = jnp.clip(ys, 0.0, H - 1.0)
        xs = jnp.clip(xs, 0.0, W - 1.0)
        y0f = jnp.floor(ys)
        x0f = jnp.floor(xs)
        return y0f.astype(jnp.int32), x0f.astype(jnp.int32), ys - y0f, xs - x0f

    l2 = lax.broadcasted_iota(jnp.int32, (bm, NROW), 1)
    bin2 = l2 // RPB
    k2 = l2 % RPB
    sub2 = k2 // 2
    xc2 = k2 % 2
    iy2 = ((bin2 // OUT) * SR + sub2 // SR).astype(jnp.float32)
    ix2 = ((bin2 % OUT) * SR + sub2 % SR).astype(jnp.float32)
    y0, x0, _, _ = sample(iy2, ix2)
    xsel = jnp.where(xc2 == 1, jnp.minimum(x0 + 1, W - 1), x0)
    idx_ref[...] = b * (H * W) + y0 * W + xsel

    l = lax.broadcasted_iota(jnp.int32, (bm, NWT), 1)
    bin_ = l // KPB
    k16 = l % KPB
    k = k16 // 2
    yc = k16 % 2
    sub = k // 2
    xc = k % 2
    iy = ((bin_ // OUT) * SR + sub // SR).astype(jnp.float32)
    ix = ((bin_ % OUT) * SR + sub % SR).astype(jnp.float32)
    _, _, ly, lx = sample(iy, ix)
    wy = jnp.where(yc == 1, ly, 1.0 - ly)
    wx = jnp.where(xc == 1, lx, 1.0 - lx)
    w_ref[...] = wy * wx * (1.0 / (SR * SR))


def _make_index_kernel(M, H, W, C):
    bm = 200 if M % 200 == 0 else M
    grid = M // bm
    return pl.pallas_call(
        functools.partial(_index_weight_body, H=H, W=W, C=C),
        grid=(grid,),
        in_specs=[pl.BlockSpec((bm, 5), lambda i: (i, 0))],
        out_specs=[
            pl.BlockSpec((bm, NROW), lambda i: (i, 0)),
            pl.BlockSpec((bm, NWT), lambda i: (i, 0)),
        ],
        out_shape=[
            jax.ShapeDtypeStruct((M, NROW), jnp.int32),
            jax.ShapeDtypeStruct((M, NWT), jnp.float32),
        ],
    )


def _make_sc_gather(M, V, C):
    info = plsc.get_sparse_core_info()
    NC, NS = info.num_cores, info.num_subcores
    NW = NC * NS
    rois_per_w = (M + NW - 1) // NW
    R8 = C // 16
    G4 = C // 32
    mesh = plsc.VectorSubcoreMesh(core_axis_name="c", subcore_axis_name="s")

    @functools.partial(
        pl.kernel,
        mesh=mesh,
        out_type=jax.ShapeDtypeStruct((M, NBIN, C), jnp.float32),
        scratch_types=[
            pltpu.VMEM((ROWS0,), jnp.int32),
            pltpu.VMEM((ROWS1,), jnp.int32),
            pltpu.VMEM((ROWS0,), jnp.int32),
            pltpu.VMEM((ROWS1,), jnp.int32),
            pltpu.VMEM((NWT,), jnp.float32),
            pltpu.VMEM((NWT,), jnp.float32),
            pltpu.VMEM((ROWS0, C), jnp.int32),
            pltpu.VMEM((ROWS1, C), jnp.int32),
            pltpu.VMEM((2, NBIN, C), jnp.float32),
            pltpu.SemaphoreType.DMA,
            pltpu.SemaphoreType.DMA,
            pltpu.SemaphoreType.DMA,
        ],
    )
    def sc_kernel(xt_hbm, idx_hbm, w_hbm, out_hbm, i00, i01, i10, i11, w_v0,
                  w_v1, rows0, rows1, out_v, isem, osem, rsem):
        wid = lax.axis_index("s") * NC + lax.axis_index("c")
        idx_refs = ((i00, i01), (i10, i11))
        w_refs = (w_v0, w_v1)
        row_bufs = (rows0, rows1)
        sizes = (ROWS0, ROWS1)

        def idx_copies(m, p):
            base = pl.multiple_of(m * NROW, 8)
            base1 = pl.multiple_of(m * NROW + ROWS0, 8)
            wbase = pl.multiple_of(m * NWT, 8)
            return (
                pltpu.make_async_copy(
                    idx_hbm.at[pl.ds(base, ROWS0)], idx_refs[p][0], isem
                ),
                pltpu.make_async_copy(
                    idx_hbm.at[pl.ds(base1, ROWS1)], idx_refs[p][1], isem
                ),
                pltpu.make_async_copy(
                    w_hbm.at[pl.ds(wbase, NWT)], w_refs[p], isem
                ),
            )

        def row_stream(p, h):
            return pltpu.make_async_copy(
                xt_hbm.at[idx_refs[p][h]], row_bufs[h], rsem
            )

        def out_copy(p, m):
            return pltpu.make_async_copy(out_v.at[p], out_hbm.at[m], osem)

        def compute_half(p, h):
            nbins = sizes[h] // RPB
            bin0 = (0, BIN0)[h]
            buf = row_bufs[h]

            def per_bin(b_, carry2):
                gbin = bin0 + b_
                wvec = w_refs[p][pl.ds(gbin * KPB, KPB)]
                base = b_ * RPB
                accs = [jnp.zeros((16,), jnp.float32) for _ in range(R8)]
                for k in range(RPB):
                    wlo = wvec[2 * k]
                    whi = wvec[2 * k + 1]
                    for g in range(G4):
                        u0 = buf[base + k, pl.ds(g * 16, 16)]
                        a0 = lax.bitcast_convert_type(u0 << 16, jnp.float32)
                        b0 = lax.bitcast_convert_type(
                            u0 & jnp.int32(-65536), jnp.float32
                        )
                        u1 = buf[base + k, pl.ds(C // 2 + g * 16, 16)]
                        a1 = lax.bitcast_convert_type(u1 << 16, jnp.float32)
                        b1 = lax.bitcast_convert_type(
                            u1 & jnp.int32(-65536), jnp.float32
                        )
                        accs[g] = accs[g] + wlo * a0 + whi * a1
                        accs[g + G4] = accs[g + G4] + wlo * b0 + whi * b1
                for r in range(R8):
                    out_v[p, gbin, pl.ds(r * 16, 16)] = accs[r]
                return carry2

            lax.fori_loop(0, nbins, per_bin, 0)

        def per_roi(j, p):
            m = j * NW + wid

            @pl.when(m < M)
            def _():
                row_stream(p, 0).wait()
                row_stream(p, 1).start()

                @pl.when(m + NW < M)
                def _():
                    for cp in idx_copies(m + NW, 1 - p):
                        cp.wait()

                @pl.when(j >= 2)
                def _():
                    out_copy(p, m).wait()

                compute_half(p, 0)
                row_stream(p, 1).wait()

                @pl.when(m + NW < M)
                def _():
                    row_stream(1 - p, 0).start()

                compute_half(p, 1)

                @pl.when(m + 2 * NW < M)
                def _():
                    for cp in idx_copies(m + 2 * NW, p):
                        cp.start()

                out_copy(p, m).start()

        def pair_body(jj, carry):
            per_roi(jj * 2, 0)
            per_roi(jj * 2 + 1, 1)
            return carry

        base0 = pl.multiple_of(wid * NROW, 8)
        base0b = pl.multiple_of(wid * NROW + ROWS0, 8)
        wbase0 = pl.multiple_of(wid * NWT, 8)
        pltpu.sync_copy(idx_hbm.at[pl.ds(base0, ROWS0)], idx_refs[0][0])
        pltpu.sync_copy(idx_hbm.at[pl.ds(base0b, ROWS1)], idx_refs[0][1])
        pltpu.sync_copy(w_hbm.at[pl.ds(wbase0, NWT)], w_refs[0])
        row_stream(0, 0).start()

        @pl.when(wid + NW < M)
        def _():
            for cp in idx_copies(wid + NW, 1):
                cp.start()

        lax.fori_loop(0, (rois_per_w + 1) // 2, pair_body, 0)

        nj = (M - 1 - wid) // NW + 1
        for t in range(2):
            @pl.when(nj > t)
            def _():
                pltpu.make_async_copy(out_v.at[0], out_hbm.at[0], osem).wait()

    return sc_kernel


def kernel(x, rois):
    B, C, H, W = x.shape
    M = rois.shape[0]
    c2 = C // 2
    xb = jnp.transpose(x, (0, 2, 3, 1)).astype(jnp.bfloat16)
    lo = lax.bitcast_convert_type(xb[..., :c2], jnp.uint16).astype(jnp.uint32)
    hi = lax.bitcast_convert_type(xb[..., c2:], jnp.uint16).astype(jnp.uint32)
    top = lax.bitcast_convert_type(lo | (hi << 16), jnp.int32)
    bot = jnp.concatenate([top[:, 1:], top[:, -1:]], axis=1)
    tbl = jnp.concatenate([top, bot], axis=-1).reshape(B * H * W, C)

    idx, w = _make_index_kernel(M, H, W, C)(rois)
    out = _make_sc_gather(M, B * H * W, C)(
        tbl, idx.reshape(M * NROW), w.reshape(M * NWT)
    )
    return jnp.transpose(out.reshape(M, OUT, OUT, C), (0, 3, 1, 2))
